# baseline (speedup 1.0000x reference)
import functools

import jax
import jax.numpy as jnp
from jax import lax
from jax.experimental import pallas as pl
from jax.experimental.pallas import tpu as pltpu
from jax.experimental.pallas import tpu_sc as plsc

N = 10000
E = 640000
D0 = 64
D1 = 128
G = 512
NCLS = 4

NC = 2
NS = 16
NW = NC * NS

ECH = 128
K = 4
EPAD = 655360
EROWS = EPAD // ECH
TRASH = N
NP = N + 16
BPW = EROWS // K // NW
BPT = EROWS // K // NS

NPW = N // NW
NTAIL = N - NPW * NW
NCH = 104
NNCH = NPW // NCH

SROWS = 624
SCH1 = 320
SCH2 = 304
STAIL = N - SROWS * NS

_F32 = jnp.float32
_I32 = jnp.int32


def _mesh():
    return plsc.VectorSubcoreMesh(
        core_axis_name="c", subcore_axis_name="s",
        num_cores=NC, num_subcores=NS)


def _wid():
    return lax.axis_index("s") * NC + lax.axis_index("c")


def _zero_buf(buf, nrows, ncols):
    z16 = jnp.zeros((16,), _F32)

    def _z(r, _):
        for cg in range(ncols // 16):
            buf[r, pl.ds(cg * 16, 16)] = z16
        return 0
    lax.fori_loop(0, nrows, _z, 0)


def _init_acc_slice(acc_sh, buf, s):
    pltpu.sync_copy(buf, acc_sh.at[pl.ds(s * SROWS, SCH1), :])
    pltpu.sync_copy(buf.at[pl.ds(0, SCH2), :],
                    acc_sh.at[pl.ds(s * SROWS + SCH1, SCH2), :])

    @pl.when(s == NS - 1)
    def _():
        pltpu.sync_copy(buf.at[pl.ds(0, STAIL), :],
                        acc_sh.at[pl.ds(SROWS * NS, STAIL), :])


def _writeback_acc_slice(acc_sh, out_slicer, buf, s):
    for base, ln in ((0, SCH1), (SCH1, SCH2)):
        pltpu.sync_copy(acc_sh.at[pl.ds(s * SROWS + base, ln), :],
                        buf.at[pl.ds(0, ln), :])
        pltpu.sync_copy(buf.at[pl.ds(0, ln), :],
                        out_slicer(s * SROWS + base, ln))

    @pl.when(s == NS - 1)
    def _():
        pltpu.sync_copy(acc_sh.at[pl.ds(SROWS * NS, STAIL), :],
                        buf.at[pl.ds(0, STAIL), :])
        pltpu.sync_copy(buf.at[pl.ds(0, STAIL), :],
                        out_slicer(SROWS * NS, STAIL))


def _edge_pipeline(nblk, rbase, src2_h, dst2_h, g_h, d,
                   sidx, didx, rows, gsem, scatter_fn, bias=None):
    def load(g, blk):
        row = rbase + blk * K
        pltpu.sync_copy(src2_h.at[pl.ds(row, K), :], sidx.at[g])
        pltpu.sync_copy(dst2_h.at[pl.ds(row, K), :], didx.at[g])
        if bias is not None:
            for k in range(K):
                for j in range(ECH // 16):
                    sl = pl.ds(j * 16, 16)
                    sidx[g, k, sl] = sidx[g, k, sl] + bias

    def fire(g):
        for k in range(K):
            pltpu.async_copy(g_h.at[sidx.at[g, k]],
                             rows.at[g, pl.ds(k * ECH, ECH), :], gsem)

    def drain(g):
        for k in range(K):
            pltpu.make_async_copy(g_h.at[pl.ds(0, ECH), :],
                                  rows.at[g, pl.ds(k * ECH, ECH), :],
                                  gsem).wait()

    def scat(g):
        for k in range(K):
            scatter_fn(rows.at[g, pl.ds(k * ECH, ECH), :], didx.at[g, k])

    load(0, 0)
    fire(0)

    def _pair(p, _):
        a = 2 * p
        load(1, a + 1)
        drain(0)
        fire(1)
        scat(0)
        load(0, a + 2)
        drain(1)
        fire(0)
        scat(1)
        return 0
    lax.fori_loop(0, nblk // 2 - 1, _pair, 0)

    load(1, nblk - 1)
    drain(0)
    fire(1)
    scat(0)
    drain(1)
    scat(1)


HR = 16


@functools.cache
def _make_emb_deg():
    @functools.partial(
        pl.kernel,
        out_type=[
            jax.ShapeDtypeStruct((N, D0), _F32),
            jax.ShapeDtypeStruct((NW, NP), _F32),
        ],
        mesh=_mesh(),
        compiler_params=pltpu.CompilerParams(use_tc_tiling_on_sc=False, needs_layout_passes=False),
        scratch_types=[
            pltpu.VMEM((NCH,), _I32),
            pltpu.VMEM((NCH, D0), _F32),
            pltpu.VMEM((2, HR, ECH), _I32),
            pltpu.VMEM((1, NP), _F32),
            pltpu.SemaphoreType.DMA,
        ],
    )
    def _emb_deg(x_h, dst2_h, emb_h, h0_h, degp_h,
                 nidx, nrows, dbuf, hist, sem):
        wid = _wid()
        ones = jnp.ones((16,), _F32)

        _zero_buf(hist, 1, NP)

        rbase = wid * (EROWS // NW)

        def fetch(g, j):
            pltpu.async_copy(
                dst2_h.at[pl.ds(rbase + j * HR, HR), :], dbuf.at[g], sem)

        def drain():
            pltpu.make_async_copy(
                dst2_h.at[pl.ds(0, HR), :], dbuf.at[0], sem).wait()

        def compute(g):
            def row(r, _):
                for c in range(ECH // 16):
                    d16 = dbuf[g, r, pl.ds(c * 16, 16)]
                    plsc.addupdate_scatter(hist.at[0], [d16], ones)
                return 0
            lax.fori_loop(0, HR, row, 0)

        nfetch = (EROWS // NW) // HR
        fetch(0, 0)
        for j in range(nfetch):
            drain()
            if j + 1 < nfetch:
                fetch((j + 1) % 2, j + 1)
            compute(j % 2)

        pltpu.sync_copy(hist.at[0], degp_h.at[wid])

        nbase = wid * NPW
        for j in range(NNCH):
            b = nbase + j * NCH
            pltpu.sync_copy(x_h.at[pl.ds(b, NCH)], nidx)
            pltpu.async_copy(emb_h.at[nidx], nrows, sem).wait()
            pltpu.sync_copy(nrows, h0_h.at[pl.ds(b, NCH), :])

        @pl.when(wid == NW - 1)
        def _tail():
            ti = nidx.at[pl.ds(0, NTAIL)]
            tr = nrows.at[pl.ds(0, NTAIL), :]
            pltpu.sync_copy(x_h.at[pl.ds(N - NTAIL, NTAIL)], ti)
            pltpu.async_copy(emb_h.at[ti], tr, sem).wait()
            pltpu.sync_copy(tr, h0_h.at[pl.ds(N - NTAIL, NTAIL), :])

    return _emb_deg


@functools.cache
def _make_spmm(d):
    @functools.partial(
        pl.kernel,
        out_type=jax.ShapeDtypeStruct((NC, N, d), _F32),
        mesh=_mesh(),
        compiler_params=pltpu.CompilerParams(use_tc_tiling_on_sc=False, needs_layout_passes=False),
        scratch_types=[
            pltpu.VMEM((2, K, ECH), _I32),
            pltpu.VMEM((2, K, ECH), _I32),
            pltpu.VMEM((2, K * ECH, d), _F32),
            pltpu.VMEM((SCH1, d), _F32),
            pltpu.VMEM_SHARED((NP, d), _F32),
            pltpu.SemaphoreType.DMA,
        ],
    )
    def _spmm(src2_h, dst2_h, g_h, pp_h,
              sidx, didx, rows, wbuf, acc_sh, gsem):
        c = lax.axis_index("c")
        s = lax.axis_index("s")
        wid = _wid()

        _zero_buf(wbuf, SCH1, d)
        _init_acc_slice(acc_sh, wbuf, s)

        plsc.subcore_barrier()

        def _scat(rows_sl, didx_row):
            pltpu.sync_copy(rows_sl, acc_sh.at[didx_row], add=True)

        _edge_pipeline(BPW, wid * BPW * K, src2_h, dst2_h, g_h, d,
                       sidx, didx, rows, gsem, _scat)

        plsc.subcore_barrier()

        _writeback_acc_slice(
            acc_sh, lambda b, ln: pp_h.at[c, pl.ds(b, ln), :], wbuf, s)

    return _spmm


@functools.cache
def _make_spmm_half():
    @functools.partial(
        pl.kernel,
        out_type=jax.ShapeDtypeStruct((NC, N, D0), _F32),
        mesh=_mesh(),
        compiler_params=pltpu.CompilerParams(use_tc_tiling_on_sc=False, needs_layout_passes=False),
        scratch_types=[
            pltpu.VMEM((2, K, ECH), _I32),
            pltpu.VMEM((2, K, ECH), _I32),
            pltpu.VMEM((2, K * ECH, D0), _F32),
            pltpu.VMEM((SCH1, D0), _F32),
            pltpu.VMEM_SHARED((NP, D0), _F32),
            pltpu.SemaphoreType.DMA,
        ],
    )
    def _spmm_h(src2_h, dst2_h, gh_h, pp_h,
                sidx, didx, rows, wbuf, acc_sh, gsem):
        c = lax.axis_index("c")
        s = lax.axis_index("s")

        _zero_buf(wbuf, SCH1, D0)
        _init_acc_slice(acc_sh, wbuf, s)

        plsc.subcore_barrier()

        bias = jnp.full((16,), c * N, dtype=_I32)

        def _scat(rows_sl, didx_row):
            pltpu.sync_copy(rows_sl, acc_sh.at[didx_row], add=True)

        _edge_pipeline(BPT, s * BPT * K, src2_h, dst2_h, gh_h, D0,
                       sidx, didx, rows, gsem, _scat, bias=bias)

        plsc.subcore_barrier()

        _writeback_acc_slice(
            acc_sh, lambda b, ln: pp_h.at[c, pl.ds(b, ln), :], wbuf, s)

    return _spmm_h


@functools.cache
def _make_pool():
    @functools.partial(
        pl.kernel,
        out_type=jax.ShapeDtypeStruct((NW, G, D1), _F32),
        mesh=_mesh(),
        compiler_params=pltpu.CompilerParams(use_tc_tiling_on_sc=False, needs_layout_passes=False),
        scratch_types=[
            pltpu.VMEM((NCH, D0), _F32),
            pltpu.VMEM((NCH, D0), _F32),
            pltpu.VMEM((NCH, D0), _F32),
            pltpu.VMEM((NCH, D0), _F32),
            pltpu.VMEM((NCH + 16,), _F32),
            pltpu.VMEM((NCH + 16,), _I32),
            pltpu.VMEM((D1,), _F32),
            pltpu.VMEM((G, D1), _F32),
        ],
    )
    def _pool(pp_h, gh_h, dinv_h, batch_h, b2_h, pools_h,
              pLb, pRb, gLb, gRb, dvb, btb, b2v, acc):
        wid = _wid()
        z16 = jnp.zeros((16,), _F32)
        iota16 = lax.iota(_I32, 16)

        def _zero(r, _):
            for cg in range(D1 // 16):
                acc[r, pl.ds(cg * 16, 16)] = z16
            return 0
        lax.fori_loop(0, G, _zero, 0)

        pltpu.sync_copy(b2_h, b2v)

        def _chunk(nb, ln):
            pltpu.sync_copy(pp_h.at[0, pl.ds(nb, ln), :],
                            pLb.at[pl.ds(0, ln), :])
            pltpu.sync_copy(pp_h.at[1, pl.ds(nb, ln), :],
                            pRb.at[pl.ds(0, ln), :])
            pltpu.sync_copy(gh_h.at[pl.ds(nb, ln), :], gLb.at[pl.ds(0, ln), :])
            pltpu.sync_copy(gh_h.at[pl.ds(N + nb, ln), :],
                            gRb.at[pl.ds(0, ln), :])
            pltpu.sync_copy(dinv_h.at[pl.ds(nb, ln)], dvb.at[pl.ds(0, ln)])
            pltpu.sync_copy(batch_h.at[pl.ds(nb, ln)], btb.at[pl.ds(0, ln)])

            def _node(i, _):
                bid = btb[pl.ds(i, 16)][0]
                dv = dvb[pl.ds(i, 16)][0]
                rowi = jnp.full((16,), bid, dtype=_I32)
                for half, pb, gb in ((0, pLb, gLb), (1, pRb, gRb)):
                    for cg in range(D0 // 16):
                        sl = pl.ds(cg * 16, 16)
                        cbase = half * D0 + cg * 16
                        v = dv * (pb[i, sl] + gb[i, sl]) + b2v[pl.ds(cbase, 16)]
                        v = jnp.maximum(v, 0.0)
                        coli = cbase + iota16
                        old = plsc.load_gather(acc, [rowi, coli])
                        plsc.store_scatter(acc, [rowi, coli],
                                           jnp.maximum(old, v))
                return 0
            lax.fori_loop(0, ln, _node, 0)

        nbase = wid * NPW
        for j in range(NNCH):
            _chunk(nbase + j * NCH, NCH)

        @pl.when(wid == NW - 1)
        def _tail():
            _chunk(N - NTAIL, NTAIL)

        pltpu.sync_copy(acc, pools_h.at[wid])

    return _pool


_RB = 1024
_NB = -(-N // _RB)


def _tc_scale(degp_ref, h0_ref, g1_ref, dinv_ref):
    deg = jnp.sum(degp_ref[...], axis=0, keepdims=True) + 1.0
    dinv = lax.rsqrt(deg).T
    g1_ref[...] = h0_ref[...] * dinv
    dinv_ref[...] = dinv


_scale_call = pl.pallas_call(
    _tc_scale,
    grid=(_NB,),
    in_specs=[
        pl.BlockSpec((NW, _RB), lambda i: (0, i)),
        pl.BlockSpec((_RB, D0), lambda i: (i, 0)),
    ],
    out_specs=[
        pl.BlockSpec((_RB, D0), lambda i: (i, 0)),
        pl.BlockSpec((_RB, 1), lambda i: (i, 0)),
    ],
    out_shape=[
        jax.ShapeDtypeStruct((N, D0), _F32),
        jax.ShapeDtypeStruct((N, 1), _F32),
    ],
)


def _tc_mats(pp_ref, g1_ref, dinv_ref, w1_ref, b1_ref, w2_ref, gh_ref):
    dinv = dinv_ref[...]
    a = (pp_ref[0] + pp_ref[1] + g1_ref[...]) * dinv
    h1 = jnp.dot(a, w1_ref[...], preferred_element_type=_F32) + b1_ref[...]
    h1 = jnp.maximum(h1, 0.0)
    g2 = jnp.dot(h1, w2_ref[...], preferred_element_type=_F32) * dinv
    gh_ref[0] = g2[:, :D0]
    gh_ref[1] = g2[:, D0:]


_mats_call = pl.pallas_call(
    _tc_mats,
    grid=(_NB,),
    in_specs=[
        pl.BlockSpec((NC, _RB, D0), lambda i: (0, i, 0)),
        pl.BlockSpec((_RB, D0), lambda i: (i, 0)),
        pl.BlockSpec((_RB, 1), lambda i: (i, 0)),
        pl.BlockSpec((D0, D1), lambda i: (0, 0)),
        pl.BlockSpec((1, D1), lambda i: (0, 0)),
        pl.BlockSpec((D1, D1), lambda i: (0, 0)),
    ],
    out_specs=pl.BlockSpec((NC, _RB, D0), lambda i: (0, i, 0)),
    out_shape=jax.ShapeDtypeStruct((NC, N, D0), _F32),
)


_GB = 128


def _tc_cls(pools_ref, wl_ref, bl_ref, out_ref):
    pooled = jnp.max(pools_ref[...], axis=0)
    out_ref[...] = (
        jnp.dot(pooled, wl_ref[...], preferred_element_type=_F32)
        + bl_ref[...])


_cls_call = pl.pallas_call(
    _tc_cls,
    grid=(G // _GB,),
    in_specs=[
        pl.BlockSpec((NW, _GB, D1), lambda i: (0, i, 0)),
        pl.BlockSpec((D1, NCLS), lambda i: (0, 0)),
        pl.BlockSpec((1, NCLS), lambda i: (0, 0)),
    ],
    out_specs=pl.BlockSpec((_GB, NCLS), lambda i: (i, 0)),
    out_shape=jax.ShapeDtypeStruct((G, NCLS), _F32),
)


def kernel(x, edge_index, batch, emb, W1, b1, W2, b2, Wl, bl):
    x = x.astype(_I32)
    src = edge_index[0].astype(_I32)
    dst = edge_index[1].astype(_I32)
    batch = batch.astype(_I32)

    pad = EPAD - E
    src2 = jnp.concatenate([src, jnp.zeros((pad,), _I32)]).reshape(EROWS, ECH)
    dst2 = jnp.concatenate([dst, jnp.full((pad,), TRASH, _I32)]
                           ).reshape(EROWS, ECH)

    h0, degp = _make_emb_deg()(x, dst2, emb)
    g1, dinv = _scale_call(degp, h0)
    pp1 = _make_spmm(D0)(src2, dst2, g1)
    gh = _mats_call(pp1, g1, dinv, W1, b1.reshape(1, D1), W2)
    gh2 = gh.reshape(NC * N, D0)
    pp2 = _make_spmm_half()(src2, dst2, gh2)
    pools = _make_pool()(pp2, gh2, dinv.reshape(-1), batch, b2)
    return _cls_call(pools, Wl, bl.reshape(1, NCLS))


# speedup vs baseline: 1.0951x; 1.0951x over previous
import functools

import jax
import jax.numpy as jnp
from jax import lax
from jax.experimental import pallas as pl
from jax.experimental.pallas import tpu as pltpu
from jax.experimental.pallas import tpu_sc as plsc

N = 10000
E = 640000
D0 = 64
D1 = 128
G = 512
NCLS = 4

NC = 2
NS = 16
NW = NC * NS

ECH = 128
K = 4
EPAD = 655360
EROWS = EPAD // ECH
TRASH = N
NP = N + 16
BPW = EROWS // K // NW
BPT = EROWS // K // NS

NPW = N // NW
NTAIL = N - NPW * NW
NCH = 104
NNCH = NPW // NCH

SROWS = 624
SCH1 = 320
SCH2 = 304
STAIL = N - SROWS * NS

_F32 = jnp.float32
_I32 = jnp.int32


def _mesh():
    return plsc.VectorSubcoreMesh(
        core_axis_name="c", subcore_axis_name="s",
        num_cores=NC, num_subcores=NS)


def _wid():
    return lax.axis_index("s") * NC + lax.axis_index("c")


def _zero_buf(buf, nrows, ncols):
    z16 = jnp.zeros((16,), _F32)

    def _z(r, _):
        for cg in range(ncols // 16):
            buf[r, pl.ds(cg * 16, 16)] = z16
        return 0
    lax.fori_loop(0, nrows, _z, 0)


def _init_acc_slice(acc_sh, buf, s):
    pltpu.sync_copy(buf, acc_sh.at[pl.ds(s * SROWS, SCH1), :])
    pltpu.sync_copy(buf.at[pl.ds(0, SCH2), :],
                    acc_sh.at[pl.ds(s * SROWS + SCH1, SCH2), :])

    @pl.when(s == NS - 1)
    def _():
        pltpu.sync_copy(buf.at[pl.ds(0, STAIL), :],
                        acc_sh.at[pl.ds(SROWS * NS, STAIL), :])


def _writeback_acc_slice(acc_sh, out_slicer, buf, s):
    for base, ln in ((0, SCH1), (SCH1, SCH2)):
        pltpu.sync_copy(acc_sh.at[pl.ds(s * SROWS + base, ln), :],
                        buf.at[pl.ds(0, ln), :])
        pltpu.sync_copy(buf.at[pl.ds(0, ln), :],
                        out_slicer(s * SROWS + base, ln))

    @pl.when(s == NS - 1)
    def _():
        pltpu.sync_copy(acc_sh.at[pl.ds(SROWS * NS, STAIL), :],
                        buf.at[pl.ds(0, STAIL), :])
        pltpu.sync_copy(buf.at[pl.ds(0, STAIL), :],
                        out_slicer(SROWS * NS, STAIL))


def _edge_pipeline(nblk, rbase, src2_h, dst2_h, g_h, d,
                   sidx, didx, rows, gsem, scatter_fn, bias=None):
    def load(g, blk):
        row = rbase + blk * K
        pltpu.sync_copy(src2_h.at[pl.ds(row, K), :], sidx.at[g])
        pltpu.sync_copy(dst2_h.at[pl.ds(row, K), :], didx.at[g])
        if bias is not None:
            for k in range(K):
                for j in range(ECH // 16):
                    sl = pl.ds(j * 16, 16)
                    sidx[g, k, sl] = sidx[g, k, sl] + bias

    def fire(g):
        for k in range(K):
            pltpu.async_copy(g_h.at[sidx.at[g, k]],
                             rows.at[g, pl.ds(k * ECH, ECH), :], gsem)

    def drain(g):
        for k in range(K):
            pltpu.make_async_copy(g_h.at[pl.ds(0, ECH), :],
                                  rows.at[g, pl.ds(k * ECH, ECH), :],
                                  gsem).wait()

    def scat(g):
        for k in range(K):
            scatter_fn(rows.at[g, pl.ds(k * ECH, ECH), :], didx.at[g, k])

    load(0, 0)
    fire(0)

    def _pair(p, _):
        a = 2 * p
        load(1, a + 1)
        drain(0)
        fire(1)
        scat(0)
        load(0, a + 2)
        drain(1)
        fire(0)
        scat(1)
        return 0
    lax.fori_loop(0, nblk // 2 - 1, _pair, 0)

    load(1, nblk - 1)
    drain(0)
    fire(1)
    scat(0)
    drain(1)
    scat(1)


HR = 16


@functools.cache
def _make_emb_deg():
    @functools.partial(
        pl.kernel,
        out_type=[
            jax.ShapeDtypeStruct((N, D0), _F32),
            jax.ShapeDtypeStruct((NW, NP), _F32),
        ],
        mesh=_mesh(),
        compiler_params=pltpu.CompilerParams(use_tc_tiling_on_sc=False, needs_layout_passes=False),
        scratch_types=[
            pltpu.VMEM((NCH,), _I32),
            pltpu.VMEM((NCH, D0), _F32),
            pltpu.VMEM((2, HR, ECH), _I32),
            pltpu.VMEM((1, NP), _F32),
            pltpu.SemaphoreType.DMA,
        ],
    )
    def _emb_deg(x_h, dst2_h, emb_h, h0_h, degp_h,
                 nidx, nrows, dbuf, hist, sem):
        wid = _wid()
        ones = jnp.ones((16,), _F32)

        _zero_buf(hist, 1, NP)

        rbase = wid * (EROWS // NW)

        def fetch(g, j):
            pltpu.async_copy(
                dst2_h.at[pl.ds(rbase + j * HR, HR), :], dbuf.at[g], sem)

        def drain():
            pltpu.make_async_copy(
                dst2_h.at[pl.ds(0, HR), :], dbuf.at[0], sem).wait()

        def compute(g):
            def row(r, _):
                for c in range(ECH // 16):
                    d16 = dbuf[g, r, pl.ds(c * 16, 16)]
                    plsc.addupdate_scatter(hist.at[0], [d16], ones)
                return 0
            lax.fori_loop(0, HR, row, 0)

        nfetch = (EROWS // NW) // HR
        fetch(0, 0)
        for j in range(nfetch):
            drain()
            if j + 1 < nfetch:
                fetch((j + 1) % 2, j + 1)
            compute(j % 2)

        pltpu.sync_copy(hist.at[0], degp_h.at[wid])

        nbase = wid * NPW
        for j in range(NNCH):
            b = nbase + j * NCH
            pltpu.sync_copy(x_h.at[pl.ds(b, NCH)], nidx)
            pltpu.async_copy(emb_h.at[nidx], nrows, sem).wait()
            pltpu.sync_copy(nrows, h0_h.at[pl.ds(b, NCH), :])

        @pl.when(wid == NW - 1)
        def _tail():
            ti = nidx.at[pl.ds(0, NTAIL)]
            tr = nrows.at[pl.ds(0, NTAIL), :]
            pltpu.sync_copy(x_h.at[pl.ds(N - NTAIL, NTAIL)], ti)
            pltpu.async_copy(emb_h.at[ti], tr, sem).wait()
            pltpu.sync_copy(tr, h0_h.at[pl.ds(N - NTAIL, NTAIL), :])

    return _emb_deg


@functools.cache
def _make_spmm(d):
    @functools.partial(
        pl.kernel,
        out_type=jax.ShapeDtypeStruct((NC, N, d), _F32),
        mesh=_mesh(),
        compiler_params=pltpu.CompilerParams(use_tc_tiling_on_sc=False, needs_layout_passes=False),
        scratch_types=[
            pltpu.VMEM((2, K, ECH), _I32),
            pltpu.VMEM((2, K, ECH), _I32),
            pltpu.VMEM((2, K * ECH, d), _F32),
            pltpu.VMEM((SCH1, d), _F32),
            pltpu.VMEM_SHARED((NP, d), _F32),
            pltpu.SemaphoreType.DMA,
        ],
    )
    def _spmm(src2_h, dst2_h, g_h, pp_h,
              sidx, didx, rows, wbuf, acc_sh, gsem):
        c = lax.axis_index("c")
        s = lax.axis_index("s")
        wid = _wid()

        _zero_buf(wbuf, SCH1, d)
        _init_acc_slice(acc_sh, wbuf, s)

        plsc.subcore_barrier()

        def _scat(rows_sl, didx_row):
            pltpu.sync_copy(rows_sl, acc_sh.at[didx_row], add=True)

        _edge_pipeline(BPW, wid * BPW * K, src2_h, dst2_h, g_h, d,
                       sidx, didx, rows, gsem, _scat)

        plsc.subcore_barrier()

        _writeback_acc_slice(
            acc_sh, lambda b, ln: pp_h.at[c, pl.ds(b, ln), :], wbuf, s)

    return _spmm


@functools.cache
def _make_spmm_half():
    @functools.partial(
        pl.kernel,
        out_type=jax.ShapeDtypeStruct((NC, N, D0), _F32),
        mesh=_mesh(),
        compiler_params=pltpu.CompilerParams(use_tc_tiling_on_sc=False, needs_layout_passes=False),
        scratch_types=[
            pltpu.VMEM((2, K, ECH), _I32),
            pltpu.VMEM((2, K, ECH), _I32),
            pltpu.VMEM((2, K * ECH, D0), _F32),
            pltpu.VMEM((SCH1, D0), _F32),
            pltpu.VMEM_SHARED((NP, D0), _F32),
            pltpu.SemaphoreType.DMA,
        ],
    )
    def _spmm_h(src2_h, dst2_h, gh_h, pp_h,
                sidx, didx, rows, wbuf, acc_sh, gsem):
        c = lax.axis_index("c")
        s = lax.axis_index("s")

        _zero_buf(wbuf, SCH1, D0)
        _init_acc_slice(acc_sh, wbuf, s)

        plsc.subcore_barrier()

        bias = jnp.full((16,), c * N, dtype=_I32)

        def _scat(rows_sl, didx_row):
            pltpu.sync_copy(rows_sl, acc_sh.at[didx_row], add=True)

        _edge_pipeline(BPT, s * BPT * K, src2_h, dst2_h, gh_h, D0,
                       sidx, didx, rows, gsem, _scat, bias=bias)

        plsc.subcore_barrier()

        _writeback_acc_slice(
            acc_sh, lambda b, ln: pp_h.at[c, pl.ds(b, ln), :], wbuf, s)

    return _spmm_h


@functools.cache
def _make_pool():
    @functools.partial(
        pl.kernel,
        out_type=jax.ShapeDtypeStruct((NW, G, D1), _F32),
        mesh=_mesh(),
        compiler_params=pltpu.CompilerParams(use_tc_tiling_on_sc=False, needs_layout_passes=False),
        scratch_types=[
            pltpu.VMEM((NCH, D0), _F32),
            pltpu.VMEM((NCH, D0), _F32),
            pltpu.VMEM((NCH, D0), _F32),
            pltpu.VMEM((NCH, D0), _F32),
            pltpu.VMEM((NCH + 16,), _F32),
            pltpu.VMEM((NCH + 16,), _I32),
            pltpu.VMEM((D1,), _F32),
            pltpu.VMEM((G, D1), _F32),
        ],
    )
    def _pool(pp_h, gh_h, dinv_h, batch_h, b2_h, pools_h,
              pLb, pRb, gLb, gRb, dvb, btb, b2v, acc):
        wid = _wid()
        z16 = jnp.zeros((16,), _F32)
        iota16 = lax.iota(_I32, 16)

        def _zero(r, _):
            for cg in range(D1 // 16):
                acc[r, pl.ds(cg * 16, 16)] = z16
            return 0
        lax.fori_loop(0, G, _zero, 0)

        pltpu.sync_copy(b2_h, b2v)

        def _chunk(nb, ln):
            pltpu.sync_copy(pp_h.at[0, pl.ds(nb, ln), :],
                            pLb.at[pl.ds(0, ln), :])
            pltpu.sync_copy(pp_h.at[1, pl.ds(nb, ln), :],
                            pRb.at[pl.ds(0, ln), :])
            pltpu.sync_copy(gh_h.at[pl.ds(nb, ln), :], gLb.at[pl.ds(0, ln), :])
            pltpu.sync_copy(gh_h.at[pl.ds(N + nb, ln), :],
                            gRb.at[pl.ds(0, ln), :])
            pltpu.sync_copy(dinv_h.at[pl.ds(nb, ln)], dvb.at[pl.ds(0, ln)])
            pltpu.sync_copy(batch_h.at[pl.ds(nb, ln)], btb.at[pl.ds(0, ln)])

            def _node(i, _):
                bid = btb[pl.ds(i, 16)][0]
                dv = dvb[pl.ds(i, 16)][0]
                rowi = jnp.full((16,), bid, dtype=_I32)
                for half, pb, gb in ((0, pLb, gLb), (1, pRb, gRb)):
                    for cg in range(D0 // 16):
                        sl = pl.ds(cg * 16, 16)
                        cbase = half * D0 + cg * 16
                        v = dv * (pb[i, sl] + gb[i, sl]) + b2v[pl.ds(cbase, 16)]
                        v = jnp.maximum(v, 0.0)
                        coli = cbase + iota16
                        old = plsc.load_gather(acc, [rowi, coli])
                        plsc.store_scatter(acc, [rowi, coli],
                                           jnp.maximum(old, v))
                return 0
            lax.fori_loop(0, ln, _node, 0)

        nbase = wid * NPW
        for j in range(NNCH):
            _chunk(nbase + j * NCH, NCH)

        @pl.when(wid == NW - 1)
        def _tail():
            _chunk(N - NTAIL, NTAIL)

        pltpu.sync_copy(acc, pools_h.at[wid])

    return _pool


_RB = 1000


def _tc_scale(degp_ref, h0_ref, g1_ref, dinv_ref):
    deg = jnp.sum(degp_ref[...], axis=1, keepdims=True) + 1.0
    dinv = lax.rsqrt(deg)
    g1_ref[...] = h0_ref[...] * dinv
    dinv_ref[...] = dinv


_scale_call = pl.pallas_call(
    _tc_scale,
    grid=(N // _RB,),
    in_specs=[
        pl.BlockSpec((_RB, NW), lambda i: (i, 0)),
        pl.BlockSpec((_RB, D0), lambda i: (i, 0)),
    ],
    out_specs=[
        pl.BlockSpec((_RB, D0), lambda i: (i, 0)),
        pl.BlockSpec((_RB, 1), lambda i: (i, 0)),
    ],
    out_shape=[
        jax.ShapeDtypeStruct((N, D0), _F32),
        jax.ShapeDtypeStruct((N, 1), _F32),
    ],
)


def _tc_mats(pp_ref, g1_ref, dinv_ref, w1_ref, b1_ref, w2_ref, gh_ref):
    dinv = dinv_ref[...]
    a = (pp_ref[0] + pp_ref[1] + g1_ref[...]) * dinv
    h1 = jnp.dot(a, w1_ref[...], preferred_element_type=_F32) + b1_ref[...]
    h1 = jnp.maximum(h1, 0.0)
    g2 = jnp.dot(h1, w2_ref[...], preferred_element_type=_F32) * dinv
    gh_ref[0] = g2[:, :D0]
    gh_ref[1] = g2[:, D0:]


_mats_call = pl.pallas_call(
    _tc_mats,
    grid=(N // _RB,),
    in_specs=[
        pl.BlockSpec((NC, _RB, D0), lambda i: (0, i, 0)),
        pl.BlockSpec((_RB, D0), lambda i: (i, 0)),
        pl.BlockSpec((_RB, 1), lambda i: (i, 0)),
        pl.BlockSpec((D0, D1), lambda i: (0, 0)),
        pl.BlockSpec((1, D1), lambda i: (0, 0)),
        pl.BlockSpec((D1, D1), lambda i: (0, 0)),
    ],
    out_specs=pl.BlockSpec((NC, _RB, D0), lambda i: (0, i, 0)),
    out_shape=jax.ShapeDtypeStruct((NC, N, D0), _F32),
)


_GB = 128


def _tc_cls(pools_ref, wl_ref, bl_ref, out_ref):
    pooled = jnp.max(pools_ref[...], axis=0)
    out_ref[...] = (
        jnp.dot(pooled, wl_ref[...], preferred_element_type=_F32)
        + bl_ref[...])


_cls_call = pl.pallas_call(
    _tc_cls,
    grid=(G // _GB,),
    in_specs=[
        pl.BlockSpec((NW, _GB, D1), lambda i: (0, i, 0)),
        pl.BlockSpec((D1, NCLS), lambda i: (0, 0)),
        pl.BlockSpec((1, NCLS), lambda i: (0, 0)),
    ],
    out_specs=pl.BlockSpec((_GB, NCLS), lambda i: (i, 0)),
    out_shape=jax.ShapeDtypeStruct((G, NCLS), _F32),
)


def kernel(x, edge_index, batch, emb, W1, b1, W2, b2, Wl, bl):
    x = x.astype(_I32)
    src = edge_index[0].astype(_I32)
    dst = edge_index[1].astype(_I32)
    batch = batch.astype(_I32)

    pad = EPAD - E
    src2 = jnp.concatenate([src, jnp.zeros((pad,), _I32)]).reshape(EROWS, ECH)
    dst2 = jnp.concatenate([dst, jnp.full((pad,), TRASH, _I32)]
                           ).reshape(EROWS, ECH)

    h0, degp = _make_emb_deg()(x, dst2, emb)
    g1, dinv = _scale_call(degp.T[:N], h0)
    pp1 = _make_spmm(D0)(src2, dst2, g1)
    gh = _mats_call(pp1, g1, dinv, W1, b1.reshape(1, D1), W2)
    gh2 = gh.reshape(NC * N, D0)
    pp2 = _make_spmm_half()(src2, dst2, gh2)
    pools = _make_pool()(pp2, gh2, dinv.reshape(-1), batch, b2)
    return _cls_call(pools, Wl, bl.reshape(1, NCLS))
